# Initial kernel scaffold; baseline (speedup 1.0000x reference)
import jax
import jax.numpy as jnp
from jax.experimental import pallas as pl


def kernel(x, edge_index, proj_W, proj_b, gat_W, att_src, att_dst, gat_b, ln_g, ln_b, cls_W1, cls_b1, cls_W2, cls_b2):
    raise NotImplementedError("write your pallas kernel here")


# speedup vs baseline: 1.0392x; 1.0392x over previous
import functools

import jax
import jax.numpy as jnp
from jax.experimental import pallas as pl

N, E, D, H, HD, L, C = 10000, 320000, 128, 8, 16, 3, 16


def _proj_body(x_ref, w_ref, b_ref, o_ref):
    o_ref[...] = jax.nn.relu(
        jnp.dot(x_ref[...], w_ref[...], preferred_element_type=jnp.float32)
        + b_ref[...]
    )


def kernel(x, edge_index, proj_W, proj_b, gat_W, att_src, att_dst, gat_b,
           ln_g, ln_b, cls_W1, cls_b1, cls_W2, cls_b2):
    src, dst = edge_index[0], edge_index[1]

    h = pl.pallas_call(
        _proj_body,
        out_shape=jax.ShapeDtypeStruct((N, D), jnp.float32),
    )(x, proj_W, proj_b.reshape(1, D))

    for i in range(L):
        h_res = h
        hw = (h @ gat_W[i]).reshape(-1, H, HD)
        a_s = (hw * att_src[i]).sum(-1)
        a_d = (hw * att_dst[i]).sum(-1)
        gmax = jnp.max(a_s, axis=0)
        m = jax.nn.leaky_relu(gmax[None, :] + a_d, 0.2)
        e = jax.nn.leaky_relu(a_s[src] + a_d[dst], 0.2)
        w = jnp.exp(e - m[dst])
        denom = jax.ops.segment_sum(w, dst, num_segments=N)
        out = jax.ops.segment_sum(hw[src] * w[:, :, None], dst, num_segments=N)
        h = out.reshape(-1, D) / (denom + 1e-16).repeat(HD, axis=1) + gat_b[i]
        h = h + h_res
        mu = jnp.mean(h, axis=-1, keepdims=True)
        var = jnp.mean((h - mu) ** 2, axis=-1, keepdims=True)
        h = ln_g[i] * (h - mu) / jnp.sqrt(var + 1e-5) + ln_b[i]
        h = jax.nn.relu(h)
    z = jax.nn.relu(h @ cls_W1 + cls_b1) @ cls_W2 + cls_b2
    return jax.nn.log_softmax(z, axis=1)


# speedup vs baseline: 74.2375x; 71.4388x over previous
import functools

import jax
import jax.numpy as jnp
from jax import lax
from jax.experimental import pallas as pl
from jax.experimental.pallas import tpu as pltpu
from jax.experimental.pallas import tpu_sc as plsc

N, E, D, H, HD, NLAYERS, C = 10000, 320000, 128, 8, 16, 3, 16
ROWW = 144
DSTW = 32
EB = 128
NROWS = E // EB
NWORK = 32
NSUB = 16
NPAD = 10240
NPT = NPAD // NSUB


def _proj_body(x_ref, w_ref, b_ref, o_ref):
    o_ref[...] = jax.nn.relu(
        jnp.dot(x_ref[...], w_ref[...], preferred_element_type=jnp.float32)
        + b_ref[...])


def _tables_body(h_ref, w_ref, as_ref, ad_ref, ts_ref, td_ref):
    hw = jnp.dot(h_ref[...], w_ref[...], preferred_element_type=jnp.float32)
    a_s = jnp.dot(hw, as_ref[...], preferred_element_type=jnp.float32)
    a_d = jnp.dot(hw, ad_ref[...], preferred_element_type=jnp.float32)
    gmax = jnp.max(a_s, axis=0, keepdims=True)
    t = gmax + a_d
    m = jnp.maximum(t, 0.2 * t)
    z8 = jnp.zeros_like(a_s)
    ts_ref[...] = jnp.concatenate([hw, a_s, z8], axis=1)
    td_ref[...] = jnp.concatenate([a_d, z8, m, z8], axis=1)


def _combine_body(p_ref, hres_ref, gb_ref, lg_ref, lb_ref, r_ref, o_ref):
    ssum = p_ref[0, :N] + p_ref[1, :N]
    out = ssum[:, 0:D]
    den = ssum[:, D:D + H]
    dexp = jnp.dot(den, r_ref[...], preferred_element_type=jnp.float32)
    h = out / (dexp + 1e-16) + gb_ref[...] + hres_ref[...]
    mu = jnp.mean(h, axis=-1, keepdims=True)
    var = jnp.mean((h - mu) ** 2, axis=-1, keepdims=True)
    h = lg_ref[...] * (h - mu) / jnp.sqrt(var + 1e-5) + lb_ref[...]
    o_ref[...] = jax.nn.relu(h)


def _cls_body(h_ref, w1_ref, b1_ref, w2_ref, b2_ref, o_ref):
    z1 = jax.nn.relu(
        jnp.dot(h_ref[...], w1_ref[...], preferred_element_type=jnp.float32)
        + b1_ref[...])
    z = jnp.dot(z1, w2_ref[...], preferred_element_type=jnp.float32) + b2_ref[...]
    zm = jnp.max(z, axis=-1, keepdims=True)
    ze = z - zm
    lse = jnp.log(jnp.sum(jnp.exp(ze), axis=-1, keepdims=True))
    o_ref[...] = ze - lse


def _edge_sc(ts_hbm, td_hbm, s2_hbm, d2_hbm, zero_hbm, out_hbm,
             sidx, didx, rows, drows, acc, gsem1, gsem2):
    c = lax.axis_index("c")
    s = lax.axis_index("s")
    wid = s * 2 + c

    pltpu.sync_copy(zero_hbm, acc.at[pl.ds(s * NPT, NPT)])
    plsc.subcore_barrier()

    base = NROWS // NWORK
    extra = NROWS - base * NWORK
    nmine = base + jnp.where(wid < extra, 1, 0)

    def batch_body(k, carry):
        r = wid + k * NWORK
        pltpu.sync_copy(s2_hbm.at[r], sidx)
        pltpu.sync_copy(d2_hbm.at[r], didx)
        cp1 = pltpu.async_copy(ts_hbm.at[sidx], rows, gsem1)
        cp2 = pltpu.async_copy(td_hbm.at[didx], drows, gsem2)
        cp1.wait()
        cp2.wait()

        def edge_body(e, carry2):
            svec = rows[e, pl.ds(D, 16)]
            advec = drows[e, pl.ds(0, 16)]
            mvec = drows[e, pl.ds(16, 16)]
            t = svec + advec
            lr = jnp.maximum(t, 0.2 * t)
            wv = jnp.exp(lr - mvec)
            lane = lax.iota(jnp.int32, 16)
            wv = jnp.where(lane < H, wv, 0.0)
            rows[e, pl.ds(D, 16)] = wv
            for h in range(H):
                w_s = wv[h]
                rows[e, pl.ds(h * HD, HD)] = rows[e, pl.ds(h * HD, HD)] * w_s
            return carry2

        lax.fori_loop(0, EB, edge_body, 0)
        pltpu.sync_copy(rows, acc.at[didx], add=True)
        return carry

    lax.fori_loop(0, nmine, batch_body, 0)
    plsc.subcore_barrier()
    pltpu.sync_copy(acc.at[pl.ds(s * NPT, NPT)],
                    out_hbm.at[c, pl.ds(s * NPT, NPT)])


_edge_call = functools.partial(
    pl.kernel,
    mesh=plsc.VectorSubcoreMesh(core_axis_name="c", subcore_axis_name="s"),
    out_type=jax.ShapeDtypeStruct((2, NPAD, ROWW), jnp.float32),
    scratch_types=[
        pltpu.VMEM((EB,), jnp.int32),
        pltpu.VMEM((EB,), jnp.int32),
        pltpu.VMEM((EB, ROWW), jnp.float32),
        pltpu.VMEM((EB, DSTW), jnp.float32),
        pltpu.VMEM_SHARED((NPAD, ROWW), jnp.float32),
        pltpu.SemaphoreType.DMA,
        pltpu.SemaphoreType.DMA,
    ],
    compiler_params=pltpu.CompilerParams(use_tc_tiling_on_sc=False),
)(_edge_sc)


def kernel(x, edge_index, proj_W, proj_b, gat_W, att_src, att_dst, gat_b,
           ln_g, ln_b, cls_W1, cls_b1, cls_W2, cls_b2):
    f32 = jnp.float32
    eye = jnp.eye(H, dtype=f32)
    As = (att_src[..., None] * eye[:, None, :]).reshape(NLAYERS, D, H)
    Ad = (att_dst[..., None] * eye[:, None, :]).reshape(NLAYERS, D, H)
    R = jnp.repeat(eye, HD, axis=1)
    src2 = edge_index[0].reshape(NROWS, EB)
    dst2 = edge_index[1].reshape(NROWS, EB)
    zeros_blk = jnp.zeros((NPT, ROWW), f32)

    h = pl.pallas_call(
        _proj_body, out_shape=jax.ShapeDtypeStruct((N, D), f32),
    )(x, proj_W, proj_b.reshape(1, D))

    for i in range(NLAYERS):
        ts, td = pl.pallas_call(
            _tables_body,
            out_shape=(jax.ShapeDtypeStruct((N, ROWW), f32),
                       jax.ShapeDtypeStruct((N, DSTW), f32)),
        )(h, gat_W[i], As[i], Ad[i])
        partial = _edge_call(ts, td, src2, dst2, zeros_blk)
        h = pl.pallas_call(
            _combine_body, out_shape=jax.ShapeDtypeStruct((N, D), f32),
        )(partial, h, gat_b[i].reshape(1, D), ln_g[i].reshape(1, D),
          ln_b[i].reshape(1, D), R)

    return pl.pallas_call(
        _cls_body, out_shape=jax.ShapeDtypeStruct((N, C), f32),
    )(h, cls_W1, cls_b1.reshape(1, D // 2), cls_W2, cls_b2.reshape(1, C))
